# baseline (speedup 1.0000x reference)
import functools

import jax
import jax.numpy as jnp
from jax.experimental import pallas as pl
from jax.experimental.pallas import tpu as pltpu


def _tap_dots(x, w_ref, ow, c, n_taps):
    n, xh = x.shape[0], x.shape[1]
    oh = xh - 2 if n_taps == 9 else xh
    m = n * oh * ow
    acc = None
    for t in range(n_taps):
        kh, kw = divmod(t, 3) if n_taps == 9 else (0, t)
        a = x[:, kh:kh + oh, kw:kw + ow, :].reshape(m, c)
        d = jnp.dot(a, w_ref[t * c:(t + 1) * c, :],
                    preferred_element_type=jnp.float32)
        acc = d if acc is None else acc + d
    return acc


def _conv_batch_kernel(x_ref, w_ref, b_ref, o_ref, *, oh, ow, c):
    n = x_ref.shape[0]
    acc = _tap_dots(x_ref[...], w_ref, ow, c, 9)
    r = jnp.maximum(acc + b_ref[...], 0.0)
    o_ref[...] = r.reshape(n, oh, ow, o_ref.shape[-1]).astype(o_ref.dtype)


def _conv_kh_kernel(x_ref, w_ref, b_ref, o_ref, acc_ref, *, oh, ow, c):
    j = pl.program_id(1)
    n = x_ref.shape[0]
    x = x_ref[:, pl.ds(j, oh), :, :]
    part = _tap_dots(x, w_ref, ow, c, 3)

    @pl.when(j == 0)
    def _():
        acc_ref[...] = part

    @pl.when(j > 0)
    def _():
        acc_ref[...] += part

    @pl.when(j == 2)
    def _():
        r = jnp.maximum(acc_ref[...] + b_ref[...], 0.0)
        o_ref[...] = r.reshape(n, oh, ow, o_ref.shape[-1]).astype(o_ref.dtype)


def _conv_tail_kernel(x_ref, w_ref, b_ref, dlw_ref, o_ref, acc_ref, *, c):
    j = pl.program_id(1)
    n = x_ref.shape[0]
    x = x_ref[:, pl.ds(j, 2), :, :]
    part = _tap_dots(x, w_ref, 2, c, 3)

    @pl.when(j == 0)
    def _():
        acc_ref[...] = part

    @pl.when(j > 0)
    def _():
        acc_ref[...] += part

    @pl.when(j == 2)
    def _():
        r = jnp.maximum(acc_ref[...] + b_ref[...], 0.0).astype(jnp.bfloat16)
        pooled = jnp.mean(
            r.reshape(n, 4, r.shape[-1]).astype(jnp.float32), axis=1)
        h_part = jnp.dot(pooled.astype(jnp.bfloat16), dlw_ref[...],
                         preferred_element_type=jnp.float32)
        o_ref[...] = h_part.reshape(o_ref.shape)


def _head_kernel(hp_ref, dlb_ref, fcw_ref, fcb_ref, o_ref):
    h = hp_ref[0] + hp_ref[1] + dlb_ref[...]
    logits = jnp.dot(h.astype(jnp.bfloat16), fcw_ref[...],
                     preferred_element_type=jnp.float32) + fcb_ref[...]
    o_ref[...] = logits


def _vmem_limit(*arrays):
    need = 2 * sum(a.size * a.dtype.itemsize for a in arrays) + (6 << 20)
    return int(min(max(need, 32 << 20), 58 << 20))


def _conv(x, w, b, *, split):
    n, h, wd, c = x.shape
    cout = w.shape[1]
    oh, ow = h - 2, wd - 2
    if split == "batch":
        nb = n // 2
        kern = functools.partial(_conv_batch_kernel, oh=oh, ow=ow, c=c)
        grid = (2,)
        sem = ("parallel",)
        in_specs = [
            pl.BlockSpec((nb, h, wd, c), lambda i: (i, 0, 0, 0)),
            pl.BlockSpec(w.shape, lambda i: (0, 0)),
            pl.BlockSpec((1, cout), lambda i: (0, 0)),
        ]
        out_spec = pl.BlockSpec((nb, oh, ow, cout), lambda i: (i, 0, 0, 0))
        scratch = []
    else:
        tn = cout // 2
        kern = functools.partial(_conv_kh_kernel, oh=oh, ow=ow, c=c)
        grid = (2, 3)
        sem = ("parallel", "arbitrary")
        in_specs = [
            pl.BlockSpec((n, h, wd, c), lambda i, j: (0, 0, 0, 0)),
            pl.BlockSpec((3 * c, tn), lambda i, j: (j, i)),
            pl.BlockSpec((1, tn), lambda i, j: (0, i)),
        ]
        out_spec = pl.BlockSpec((n, oh, ow, tn), lambda i, j: (0, 0, 0, i))
        scratch = [pltpu.VMEM((n * oh * ow, tn), jnp.float32)]
    return pl.pallas_call(
        kern,
        out_shape=jax.ShapeDtypeStruct((n, oh, ow, cout), jnp.bfloat16),
        grid=grid,
        in_specs=in_specs,
        out_specs=out_spec,
        scratch_shapes=scratch,
        compiler_params=pltpu.CompilerParams(
            dimension_semantics=sem,
            vmem_limit_bytes=_vmem_limit(x, w, b)),
    )(x, w, b)


def _tail(x, w, b, dl_w, dl_b, fc_w, fc_b):
    n, h, wd, c = x.shape
    cout = w.shape[1]
    tn = cout // 2
    nh = dl_w.shape[1]
    h_parts = pl.pallas_call(
        functools.partial(_conv_tail_kernel, c=c),
        out_shape=jax.ShapeDtypeStruct((2, n, nh), jnp.float32),
        grid=(2, 3),
        in_specs=[
            pl.BlockSpec((n, h, wd, c), lambda i, j: (0, 0, 0, 0)),
            pl.BlockSpec((3 * c, tn), lambda i, j: (j, i)),
            pl.BlockSpec((1, tn), lambda i, j: (0, i)),
            pl.BlockSpec((tn, nh), lambda i, j: (i, 0)),
        ],
        out_specs=pl.BlockSpec((1, n, nh), lambda i, j: (i, 0, 0)),
        scratch_shapes=[pltpu.VMEM((n * 4, tn), jnp.float32)],
        compiler_params=pltpu.CompilerParams(
            dimension_semantics=("parallel", "arbitrary"),
            vmem_limit_bytes=_vmem_limit(x, w, dl_w)),
    )(x, w, b, dl_w)
    logits = pl.pallas_call(
        _head_kernel,
        out_shape=jax.ShapeDtypeStruct((n, fc_w.shape[1]), jnp.float32),
        in_specs=[pl.BlockSpec(memory_space=pltpu.MemorySpace.VMEM)] * 4,
        out_specs=pl.BlockSpec(memory_space=pltpu.MemorySpace.VMEM),
    )(h_parts, dl_b, fc_w, fc_b)
    return logits


def kernel(x, conv1_w, conv1_b, conv2_w, conv2_b, conv3_w, conv3_b,
           conv4_w, conv4_b, conv5_w, conv5_b, conv6_w, conv6_b,
           conv7_w, conv7_b, conv8_w, conv8_b, dl_w, dl_b, fc_w, fc_b):
    xh = jnp.transpose(x, (0, 2, 3, 1)).astype(jnp.bfloat16)
    cin = xh.shape[-1]
    cpad = 384
    xh = jnp.pad(xh, ((0, 0), (0, 0), (0, 0), (0, cpad - cin)))
    w1 = conv1_w[:9 * cin].reshape(9, cin, conv1_w.shape[1])
    w1 = jnp.pad(w1, ((0, 0), (0, cpad - cin), (0, 0)))
    w1 = w1.reshape(9 * cpad, conv1_w.shape[1])

    h = _conv(xh, w1, conv1_b, split="batch")
    h = _conv(h, conv2_w, conv2_b, split="batch")
    h = _conv(h, conv3_w, conv3_b, split="batch")
    h = _conv(h, conv4_w, conv4_b, split="cout")
    h = _conv(h, conv5_w, conv5_b, split="cout")
    h = _conv(h, conv6_w, conv6_b, split="cout")
    h = _conv(h, conv7_w, conv7_b, split="cout")
    logits = _tail(h, conv8_w, conv8_b, dl_w, dl_b, fc_w, fc_b)
    return logits[:, :2]
